# baseline (speedup 1.0000x reference)
import functools

import jax
import jax.numpy as jnp
from jax import lax
from jax.experimental import pallas as pl
from jax.experimental.pallas import tpu as pltpu
from jax.experimental.pallas import tpu_sc as plsc

_BATCH = 16384
_D = 64
_NC = 2
_NS = 16
_NW = _NC * _NS
_BPW = _BATCH // _NW
_C = 128
_NCHUNK = _BPW // _C
_NBUF = 2
_L = 16
_GROUPS = _C // _L


def _body(s_hbm, r_hbm, o_hbm, er_hbm, ei_hbm, rr_hbm, ri_hbm, out_hbm,
          idx_s, idx_r, idx_o, rows, red, out_v, sems, out_sem):
    wid = lax.axis_index("s") * _NC + lax.axis_index("c")
    base = wid * _BPW

    pltpu.sync_copy(s_hbm.at[pl.ds(base, _BPW)], idx_s)
    pltpu.sync_copy(r_hbm.at[pl.ds(base, _BPW)], idx_r)
    pltpu.sync_copy(o_hbm.at[pl.ds(base, _BPW)], idx_o)

    def fire(c, slot):
        lo = c * _C
        return [
            pltpu.async_copy(er_hbm.at[idx_s.at[pl.ds(lo, _C)]],
                             rows.at[slot, 0], sems.at[slot]),
            pltpu.async_copy(ei_hbm.at[idx_s.at[pl.ds(lo, _C)]],
                             rows.at[slot, 1], sems.at[slot]),
            pltpu.async_copy(er_hbm.at[idx_o.at[pl.ds(lo, _C)]],
                             rows.at[slot, 2], sems.at[slot]),
            pltpu.async_copy(ei_hbm.at[idx_o.at[pl.ds(lo, _C)]],
                             rows.at[slot, 3], sems.at[slot]),
            pltpu.async_copy(rr_hbm.at[idx_r.at[pl.ds(lo, _C)]],
                             rows.at[slot, 4], sems.at[slot]),
            pltpu.async_copy(ri_hbm.at[idx_r.at[pl.ds(lo, _C)]],
                             rows.at[slot, 5], sems.at[slot]),
        ]

    lane = lax.iota(jnp.int32, _L)
    tcol = lane * 17

    def compute(slot, c):
        def group(g, _):
            for e in range(_L):
                row = g * _L + e
                acc = None
                for j in range(_D // _L):
                    dsl = pl.ds(j * _L, _L)
                    sre = rows[slot, 0, row, dsl]
                    sim = rows[slot, 1, row, dsl]
                    ore = rows[slot, 2, row, dsl]
                    oim = rows[slot, 3, row, dsl]
                    rre = rows[slot, 4, row, dsl]
                    rim = rows[slot, 5, row, dsl]
                    t = rre * (sre * ore + sim * oim) + rim * (sre * oim - sim * ore)
                    acc = t if acc is None else acc + t
                red[pl.ds(e * 17, _L)] = acc
            scores = plsc.load_gather(red, [tcol])
            for l in range(1, _L):
                scores = scores + plsc.load_gather(red, [tcol + l])
            scores = 1.0 / (1.0 + jnp.exp(-scores))
            out_v[pl.ds(g * _L, _L)] = scores
            return _

        lax.fori_loop(0, _GROUPS, group, 0)
        pltpu.sync_copy(out_v, out_hbm.at[pl.ds(base + c * _C, _C)])

    handles = fire(0, 0)
    for c in range(_NCHUNK):
        nxt = None
        if c + 1 < _NCHUNK:
            nxt = fire(c + 1, (c + 1) % _NBUF)
        for h in handles:
            h.wait()
        compute(c % _NBUF, c)
        handles = nxt


@jax.jit
def kernel(s, r, o, entity_real, entity_imag, relation_real, relation_imag):
    mesh = plsc.VectorSubcoreMesh(core_axis_name="c", subcore_axis_name="s")
    f = pl.kernel(
        _body,
        out_type=jax.ShapeDtypeStruct((_BATCH,), jnp.float32),
        mesh=mesh,
        scratch_types=[
            pltpu.VMEM((_BPW,), jnp.int32),
            pltpu.VMEM((_BPW,), jnp.int32),
            pltpu.VMEM((_BPW,), jnp.int32),
            pltpu.VMEM((_NBUF, 6, _C, _D), jnp.float32),
            pltpu.VMEM((16 * 17,), jnp.float32),
            pltpu.VMEM((_C,), jnp.float32),
            pltpu.SemaphoreType.DMA((_NBUF,)),
            pltpu.SemaphoreType.DMA,
        ],
        compiler_params=pltpu.CompilerParams(
            needs_layout_passes=False, use_tc_tiling_on_sc=False),
    )
    return f(s, r, o, entity_real, entity_imag, relation_real, relation_imag)


# speedup vs baseline: 1.2106x; 1.2106x over previous
import functools

import jax
import jax.numpy as jnp
from jax import lax
from jax.experimental import pallas as pl
from jax.experimental.pallas import tpu as pltpu
from jax.experimental.pallas import tpu_sc as plsc

_BATCH = 16384
_D = 64
_NC = 2
_NS = 16
_NW = _NC * _NS
_BPW = _BATCH // _NW
_C = 128
_NCHUNK = _BPW // _C
_NBUF = 2
_L = 16
_GROUPS = _C // _L


def _body(s_hbm, r_hbm, o_hbm, ent_hbm, rel_hbm, out_hbm,
          idx_s, idx_r, idx_o, srows, orows, rrows, red, out_v, sems):
    wid = lax.axis_index("s") * _NC + lax.axis_index("c")
    base = wid * _BPW

    pltpu.sync_copy(s_hbm.at[pl.ds(base, _BPW)], idx_s)
    pltpu.sync_copy(r_hbm.at[pl.ds(base, _BPW)], idx_r)
    pltpu.sync_copy(o_hbm.at[pl.ds(base, _BPW)], idx_o)

    def fire(c, slot):
        lo = c * _C
        return [
            pltpu.async_copy(ent_hbm.at[idx_s.at[pl.ds(lo, _C)]],
                             srows.at[slot], sems.at[slot]),
            pltpu.async_copy(ent_hbm.at[idx_o.at[pl.ds(lo, _C)]],
                             orows.at[slot], sems.at[slot]),
            pltpu.async_copy(rel_hbm.at[idx_r.at[pl.ds(lo, _C)]],
                             rrows.at[slot], sems.at[slot]),
        ]

    lane = lax.iota(jnp.int32, _L)
    tcol = lane * 17

    def compute(slot, c):
        def group(g, _):
            for e in range(_L):
                row = g * _L + e
                acc = None
                for j in range(_D // _L):
                    dre = pl.ds(j * _L, _L)
                    dim = pl.ds(_D + j * _L, _L)
                    sre = srows[slot, row, dre]
                    sim = srows[slot, row, dim]
                    ore = orows[slot, row, dre]
                    oim = orows[slot, row, dim]
                    rre = rrows[slot, row, dre]
                    rim = rrows[slot, row, dim]
                    t = rre * (sre * ore + sim * oim) + rim * (sre * oim - sim * ore)
                    acc = t if acc is None else acc + t
                red[pl.ds(e * 17, _L)] = acc
            scores = plsc.load_gather(red, [tcol])
            for l in range(1, _L):
                scores = scores + plsc.load_gather(red, [tcol + l])
            scores = 1.0 / (1.0 + jnp.exp(-scores))
            out_v[pl.ds(g * _L, _L)] = scores
            return _

        lax.fori_loop(0, _GROUPS, group, 0)
        pltpu.sync_copy(out_v, out_hbm.at[pl.ds(base + c * _C, _C)])

    handles = fire(0, 0)
    for c in range(_NCHUNK):
        nxt = None
        if c + 1 < _NCHUNK:
            nxt = fire(c + 1, (c + 1) % _NBUF)
        for h in handles:
            h.wait()
        compute(c % _NBUF, c)
        handles = nxt


@jax.jit
def kernel(s, r, o, entity_real, entity_imag, relation_real, relation_imag):
    ent = jnp.concatenate([entity_real, entity_imag], axis=1)
    rel = jnp.concatenate([relation_real, relation_imag], axis=1)
    mesh = plsc.VectorSubcoreMesh(core_axis_name="c", subcore_axis_name="s")
    f = pl.kernel(
        _body,
        out_type=jax.ShapeDtypeStruct((_BATCH,), jnp.float32),
        mesh=mesh,
        scratch_types=[
            pltpu.VMEM((_BPW,), jnp.int32),
            pltpu.VMEM((_BPW,), jnp.int32),
            pltpu.VMEM((_BPW,), jnp.int32),
            pltpu.VMEM((_NBUF, _C, 2 * _D), jnp.float32),
            pltpu.VMEM((_NBUF, _C, 2 * _D), jnp.float32),
            pltpu.VMEM((_NBUF, _C, 2 * _D), jnp.float32),
            pltpu.VMEM((16 * 17,), jnp.float32),
            pltpu.VMEM((_C,), jnp.float32),
            pltpu.SemaphoreType.DMA((_NBUF,)),
        ],
        compiler_params=pltpu.CompilerParams(
            needs_layout_passes=False, use_tc_tiling_on_sc=False),
    )
    return f(s, r, o, ent, rel)


# speedup vs baseline: 2.3357x; 1.9294x over previous
import functools

import jax
import jax.numpy as jnp
from jax import lax
from jax.experimental import pallas as pl
from jax.experimental.pallas import tpu as pltpu
from jax.experimental.pallas import tpu_sc as plsc

_BATCH = 16384
_D = 64
_NC = 2
_NS = 16
_NW = _NC * _NS
_BPW = _BATCH // _NW
_C = 128
_NCHUNK = _BPW // _C
_NBUF = 2
_L = 16
_GROUPS = _C // _L


def _body(s_hbm, r_hbm, o_hbm, ent_hbm, rel_hbm, out_hbm,
          idx_s, idx_r, idx_o, srows, orows, rrows, red, out_v, sems):
    wid = lax.axis_index("s") * _NC + lax.axis_index("c")
    base = wid * _BPW

    pltpu.sync_copy(s_hbm.at[pl.ds(base, _BPW)], idx_s)
    pltpu.sync_copy(r_hbm.at[pl.ds(base, _BPW)], idx_r)
    pltpu.sync_copy(o_hbm.at[pl.ds(base, _BPW)], idx_o)

    def fire(c, slot):
        lo = c * _C
        return [
            pltpu.async_copy(ent_hbm.at[idx_s.at[pl.ds(lo, _C)]],
                             srows.at[slot], sems.at[slot]),
            pltpu.async_copy(ent_hbm.at[idx_o.at[pl.ds(lo, _C)]],
                             orows.at[slot], sems.at[slot]),
            pltpu.async_copy(rel_hbm.at[idx_r.at[pl.ds(lo, _C)]],
                             rrows.at[slot], sems.at[slot]),
        ]

    lane = lax.iota(jnp.int32, _L)
    tcol = lane * 17

    def compute(slot, c):
        def group(g, _):
            for e in range(_L):
                row = g * _L + e
                acc = None
                for j in range(_D // _L):
                    dre = pl.ds(j * _L, _L)
                    dim = pl.ds(_D + j * _L, _L)
                    sre = srows[slot, row, dre]
                    sim = srows[slot, row, dim]
                    ore = orows[slot, row, dre]
                    oim = orows[slot, row, dim]
                    rre = rrows[slot, row, dre]
                    rim = rrows[slot, row, dim]
                    t = rre * (sre * ore + sim * oim) + rim * (sre * oim - sim * ore)
                    acc = t if acc is None else acc + t
                red[pl.ds(e * 17, _L)] = acc
            scores = plsc.load_gather(red, [tcol])
            for l in range(1, _L):
                scores = scores + plsc.load_gather(red, [tcol + l])
            scores = 1.0 / (1.0 + jnp.exp(-scores))
            out_v[pl.ds(g * _L, _L)] = scores
            return _

        lax.fori_loop(0, _GROUPS, group, 0)
        pltpu.sync_copy(out_v, out_hbm.at[pl.ds(base + c * _C, _C)])

    handles = fire(0, 0)
    for c in range(_NCHUNK):
        nxt = None
        if c + 1 < _NCHUNK:
            nxt = fire(c + 1, (c + 1) % _NBUF)
        for h in handles:
            h.wait()
        compute(c % _NBUF, c)
        handles = nxt


_NE = 1000000
_TBLK = 8192
_TGRID = (_NE + _TBLK - 1) // _TBLK


def _transpose_body(re_ref, im_ref, out_ref):
    out_ref[:, 0:_D] = jnp.transpose(re_ref[...], (1, 0))
    out_ref[:, _D:2 * _D] = jnp.transpose(im_ref[...], (1, 0))


def _fuse_tables(et_re_t, et_im_t):
    return pl.pallas_call(
        _transpose_body,
        grid=(_TGRID,),
        in_specs=[
            pl.BlockSpec((_D, _TBLK), lambda k: (0, k)),
            pl.BlockSpec((_D, _TBLK), lambda k: (0, k)),
        ],
        out_specs=pl.BlockSpec((_TBLK, 2 * _D), lambda k: (k, 0)),
        out_shape=jax.ShapeDtypeStruct((_NE, 2 * _D), jnp.float32),
    )(et_re_t, et_im_t)


@jax.jit
def kernel(s, r, o, entity_real, entity_imag, relation_real, relation_imag):
    ent = _fuse_tables(entity_real.T, entity_imag.T)
    rel = jnp.concatenate([relation_real, relation_imag], axis=1)
    mesh = plsc.VectorSubcoreMesh(core_axis_name="c", subcore_axis_name="s")
    f = pl.kernel(
        _body,
        out_type=jax.ShapeDtypeStruct((_BATCH,), jnp.float32),
        mesh=mesh,
        scratch_types=[
            pltpu.VMEM((_BPW,), jnp.int32),
            pltpu.VMEM((_BPW,), jnp.int32),
            pltpu.VMEM((_BPW,), jnp.int32),
            pltpu.VMEM((_NBUF, _C, 2 * _D), jnp.float32),
            pltpu.VMEM((_NBUF, _C, 2 * _D), jnp.float32),
            pltpu.VMEM((_NBUF, _C, 2 * _D), jnp.float32),
            pltpu.VMEM((16 * 17,), jnp.float32),
            pltpu.VMEM((_C,), jnp.float32),
            pltpu.SemaphoreType.DMA((_NBUF,)),
        ],
        compiler_params=pltpu.CompilerParams(
            needs_layout_passes=False, use_tc_tiling_on_sc=False),
    )
    return f(s, r, o, ent, rel)


# speedup vs baseline: 3.0531x; 1.3071x over previous
import functools

import jax
import jax.numpy as jnp
from jax import lax
from jax.experimental import pallas as pl
from jax.experimental.pallas import tpu as pltpu
from jax.experimental.pallas import tpu_sc as plsc

_BATCH = 16384
_D = 64
_NC = 2
_NS = 16
_NW = _NC * _NS
_BPW = _BATCH // _NW
_C = 128
_NCHUNK = _BPW // _C
_NBUF = 2
_L = 16
_GROUPS = _C // _L


def _body(s_hbm, r_hbm, o_hbm, ent_hbm, rel_hbm, out_hbm,
          idx_s, idx_r, idx_o, srows, orows, rrows, red, out_v, sems):
    wid = lax.axis_index("s") * _NC + lax.axis_index("c")
    base = wid * _BPW

    pltpu.sync_copy(s_hbm.at[pl.ds(base, _BPW)], idx_s)
    pltpu.sync_copy(r_hbm.at[pl.ds(base, _BPW)], idx_r)
    pltpu.sync_copy(o_hbm.at[pl.ds(base, _BPW)], idx_o)

    def fire(c, slot):
        lo = c * _C
        return [
            pltpu.async_copy(ent_hbm.at[idx_s.at[pl.ds(lo, _C)]],
                             srows.at[slot], sems.at[slot]),
            pltpu.async_copy(ent_hbm.at[idx_o.at[pl.ds(lo, _C)]],
                             orows.at[slot], sems.at[slot]),
            pltpu.async_copy(rel_hbm.at[idx_r.at[pl.ds(lo, _C)]],
                             rrows.at[slot], sems.at[slot]),
        ]

    lane = lax.iota(jnp.int32, _L)
    tcol = lane * 17

    def compute(slot, c):
        def group(g, _):
            for e in range(_L):
                row = g * _L + e
                acc = None
                for j in range(_D // _L):
                    dre = pl.ds(j * _L, _L)
                    dim = pl.ds(_D + j * _L, _L)
                    sre = srows[slot, row, dre]
                    sim = srows[slot, row, dim]
                    ore = orows[slot, row, dre]
                    oim = orows[slot, row, dim]
                    rre = rrows[slot, row, dre]
                    rim = rrows[slot, row, dim]
                    t = rre * (sre * ore + sim * oim) + rim * (sre * oim - sim * ore)
                    acc = t if acc is None else acc + t
                red[pl.ds(e * 17, _L)] = acc
            scores = plsc.load_gather(red, [tcol])
            for l in range(1, _L):
                scores = scores + plsc.load_gather(red, [tcol + l])
            scores = 1.0 / (1.0 + jnp.exp(-scores))
            out_v[pl.ds(g * _L, _L)] = scores
            return _

        lax.fori_loop(0, _GROUPS, group, 0)
        pltpu.sync_copy(out_v, out_hbm.at[pl.ds(base + c * _C, _C)])

    handles = fire(0, 0)
    for c in range(_NCHUNK):
        nxt = None
        if c + 1 < _NCHUNK:
            nxt = fire(c + 1, (c + 1) % _NBUF)
        for h in handles:
            h.wait()
        compute(c % _NBUF, c)
        handles = nxt


_NE = 1000000
_TBLK = 8192
_TGRID = (_NE + _TBLK - 1) // _TBLK


def _transpose_body(re_ref, im_ref, out_ref):
    x = jnp.concatenate([re_ref[...], im_ref[...]], axis=0)
    out_ref[...] = jnp.transpose(x, (1, 0))


def _fuse_tables(et_re_t, et_im_t):
    return pl.pallas_call(
        _transpose_body,
        grid=(_TGRID,),
        in_specs=[
            pl.BlockSpec((_D, _TBLK), lambda k: (0, k)),
            pl.BlockSpec((_D, _TBLK), lambda k: (0, k)),
        ],
        out_specs=pl.BlockSpec((_TBLK, 2 * _D), lambda k: (k, 0)),
        out_shape=jax.ShapeDtypeStruct((_NE, 2 * _D), jnp.float32),
    )(et_re_t, et_im_t)


@jax.jit
def kernel(s, r, o, entity_real, entity_imag, relation_real, relation_imag):
    ent = _fuse_tables(entity_real.T, entity_imag.T)
    rel = jnp.concatenate([relation_real, relation_imag], axis=1)
    mesh = plsc.VectorSubcoreMesh(core_axis_name="c", subcore_axis_name="s")
    f = pl.kernel(
        _body,
        out_type=jax.ShapeDtypeStruct((_BATCH,), jnp.float32),
        mesh=mesh,
        scratch_types=[
            pltpu.VMEM((_BPW,), jnp.int32),
            pltpu.VMEM((_BPW,), jnp.int32),
            pltpu.VMEM((_BPW,), jnp.int32),
            pltpu.VMEM((_NBUF, _C, 2 * _D), jnp.float32),
            pltpu.VMEM((_NBUF, _C, 2 * _D), jnp.float32),
            pltpu.VMEM((_NBUF, _C, 2 * _D), jnp.float32),
            pltpu.VMEM((16 * 17,), jnp.float32),
            pltpu.VMEM((_C,), jnp.float32),
            pltpu.SemaphoreType.DMA((_NBUF,)),
        ],
        compiler_params=pltpu.CompilerParams(
            needs_layout_passes=False, use_tc_tiling_on_sc=False),
    )
    return f(s, r, o, ent, rel)
